# Initial kernel scaffold; baseline (speedup 1.0000x reference)
import jax
import jax.numpy as jnp
from jax.experimental import pallas as pl


def kernel(x, values, indices_float):
    raise NotImplementedError("write your pallas kernel here")


# speedup vs baseline: 7.5893x; 7.5893x over previous
import functools

import jax
import jax.numpy as jnp
from jax import lax
from jax.experimental import pallas as pl
from jax.experimental.pallas import tpu as pltpu
from jax.experimental.pallas import tpu_sc as plsc

NC = 2
NS = 16
NL = 16
NW = NC * NS

CHUNK = 128


def _spmm_sc_kernel(cout, n_chunks, xt_hbm, cols_hbm, rows_hbm, vals_hbm,
                    out_hbm, cols_v, rows_v, vals_v, g_v, accum, sem):
    c = lax.axis_index("c")
    s = lax.axis_index("s")
    wid = c * NS + s
    b = g_v.shape[1]

    zero16 = jnp.zeros((NL,), jnp.float32)

    def _zrow(i, _):
        for q in range(b // NL):
            g_v[i, pl.ds(q * NL, NL)] = zero16
        return 0
    lax.fori_loop(0, CHUNK, _zrow, 0)

    rows_per_tile = cout // NS
    for k in range(rows_per_tile // CHUNK):
        pltpu.sync_copy(g_v, accum.at[pl.ds(s * rows_per_tile + k * CHUNK, CHUNK)])
    plsc.subcore_barrier()

    pltpu.sync_copy(cols_hbm.at[wid], cols_v)
    pltpu.sync_copy(rows_hbm.at[wid], rows_v)
    pltpu.sync_copy(vals_hbm.at[wid], vals_v)

    def _chunk_body(ch, _):
        pltpu.async_copy(xt_hbm.at[cols_v.at[ch]], g_v, sem).wait()

        base16 = jnp.full((NL,), ch * CHUNK, jnp.int32)

        def _scale(j, _):
            v16 = plsc.load_gather(vals_v, [base16 + j])
            for q in range(b // NL):
                g_v[j, pl.ds(q * NL, NL)] = g_v[j, pl.ds(q * NL, NL)] * v16
            return 0
        lax.fori_loop(0, CHUNK, _scale, 0)

        pltpu.sync_copy(g_v, accum.at[rows_v.at[ch]], add=True)
        return 0

    lax.fori_loop(0, n_chunks, _chunk_body, 0)
    plsc.subcore_barrier()

    pltpu.sync_copy(accum.at[pl.ds(s * rows_per_tile, rows_per_tile)],
                    out_hbm.at[c].at[pl.ds(s * rows_per_tile, rows_per_tile)])


def _merge_body(p_ref, o_ref):
    o_ref[...] = p_ref[0] + p_ref[1]


@jax.jit
def kernel(x, values, indices_float):
    b, cin = x.shape
    nnz = values.shape[0]
    cout = cin

    idx = jnp.round(indices_float).astype(jnp.int32)
    rows, cols = idx[0], idx[1]
    xt = x.T

    per_tile = -(-nnz // NW)
    per_tile = -(-per_tile // CHUNK) * CHUNK
    n_chunks = per_tile // CHUNK
    pad = NW * per_tile - nnz
    rows_p = jnp.pad(rows, (0, pad)).reshape(NW, n_chunks, CHUNK)
    cols_p = jnp.pad(cols, (0, pad)).reshape(NW, n_chunks, CHUNK)
    vals_p = jnp.pad(values, (0, pad)).reshape(NW, n_chunks * CHUNK)

    spmm = functools.partial(
        pl.kernel,
        out_type=jax.ShapeDtypeStruct((NC, cout, b), jnp.float32),
        mesh=plsc.VectorSubcoreMesh(core_axis_name="c", subcore_axis_name="s"),
        scratch_types=[
            pltpu.VMEM((n_chunks, CHUNK), jnp.int32),
            pltpu.VMEM((n_chunks, CHUNK), jnp.int32),
            pltpu.VMEM((n_chunks * CHUNK,), jnp.float32),
            pltpu.VMEM((CHUNK, b), jnp.float32),
            pltpu.VMEM_SHARED((cout, b), jnp.float32),
            pltpu.SemaphoreType.DMA,
        ],
        compiler_params=pltpu.CompilerParams(needs_layout_passes=False,
                                             use_tc_tiling_on_sc=False),
    )(functools.partial(_spmm_sc_kernel, cout, n_chunks))

    partials = spmm(xt, cols_p, rows_p, vals_p)

    n_blk = 8
    merged = pl.pallas_call(
        _merge_body,
        out_shape=jax.ShapeDtypeStruct((cout, b), jnp.float32),
        grid=(n_blk,),
        in_specs=[pl.BlockSpec((NC, cout // n_blk, b), lambda i: (0, i, 0))],
        out_specs=pl.BlockSpec((cout // n_blk, b), lambda i: (i, 0)),
    )(partials)
    return merged.T


# speedup vs baseline: 9.1759x; 1.2091x over previous
import functools

import jax
import jax.numpy as jnp
from jax import lax
from jax.experimental import pallas as pl
from jax.experimental.pallas import tpu as pltpu
from jax.experimental.pallas import tpu_sc as plsc

NC = 2
NS = 16
NL = 16
NW = NC * NS

CHUNK = 128


def _spmm_sc_kernel(cout, n_chunks, xt_hbm, cols_hbm, rows_hbm, vals_hbm,
                    out_hbm, cols_v, rows_v, vals_v, g_a, g_b,
                    accum, sem_ga, sem_gb, sem_sa, sem_sb):
    c = lax.axis_index("c")
    s = lax.axis_index("s")
    wid = c * NS + s
    b = g_a.shape[1]

    zero16 = jnp.zeros((NL,), jnp.float32)

    def _zrow(i, _):
        for q in range(b // NL):
            g_a[i, pl.ds(q * NL, NL)] = zero16
        return 0
    lax.fori_loop(0, CHUNK, _zrow, 0)

    rows_per_tile = cout // NS
    for k in range(rows_per_tile // CHUNK):
        pltpu.sync_copy(g_a, accum.at[pl.ds(s * rows_per_tile + k * CHUNK, CHUNK)])
    plsc.subcore_barrier()

    pltpu.sync_copy(cols_hbm.at[wid], cols_v)
    pltpu.sync_copy(rows_hbm.at[wid], rows_v)
    pltpu.sync_copy(vals_hbm.at[wid], vals_v)

    def _gather(ch, g, sem):
        pltpu.async_copy(xt_hbm.at[cols_v.at[ch]], g, sem)

    def _gather_wait(ch, g, sem):
        pltpu.make_async_copy(xt_hbm.at[cols_v.at[ch]], g, sem).wait()

    def _scatter(ch, g, sem):
        pltpu.async_copy(g, accum.at[rows_v.at[ch]], sem, add=True)

    def _scatter_wait(ch, g, sem):
        pltpu.make_async_copy(g, accum.at[rows_v.at[ch]], sem).wait()

    def _scale(ch, g):
        base16 = jnp.full((NL,), ch * CHUNK, jnp.int32)

        def _body(j, _):
            v16 = plsc.load_gather(vals_v, [base16 + j])
            for q in range(b // NL):
                g[j, pl.ds(q * NL, NL)] = g[j, pl.ds(q * NL, NL)] * v16
            return 0
        lax.fori_loop(0, CHUNK, _body, 0)

    _gather(0, g_a, sem_ga)
    _gather(1, g_b, sem_gb)
    last = n_chunks - 1

    def _chunk_body(i, _):
        c0 = 2 * i
        c1 = 2 * i + 1
        _gather_wait(c0, g_a, sem_ga)
        _scale(c0, g_a)
        _scatter(c0, g_a, sem_sa)
        _gather_wait(c1, g_b, sem_gb)
        _scale(c1, g_b)
        _scatter(c1, g_b, sem_sb)
        _scatter_wait(c0, g_a, sem_sa)
        _gather(jnp.minimum(c0 + 2, last), g_a, sem_ga)
        _scatter_wait(c1, g_b, sem_sb)
        _gather(jnp.minimum(c1 + 2, last), g_b, sem_gb)
        return 0

    lax.fori_loop(0, n_chunks // 2, _chunk_body, 0)
    _gather_wait(last, g_a, sem_ga)
    _gather_wait(last, g_b, sem_gb)
    plsc.subcore_barrier()

    pltpu.sync_copy(accum.at[pl.ds(s * rows_per_tile, rows_per_tile)],
                    out_hbm.at[c].at[pl.ds(s * rows_per_tile, rows_per_tile)])


def _merge_body(p_ref, o_ref):
    o_ref[...] = p_ref[0] + p_ref[1]


@jax.jit
def kernel(x, values, indices_float):
    b, cin = x.shape
    nnz = values.shape[0]
    cout = cin

    idx = jnp.round(indices_float).astype(jnp.int32)
    rows, cols = idx[0], idx[1]
    xt = x.T

    per_tile = -(-nnz // NW)
    per_tile = -(-per_tile // CHUNK) * CHUNK
    n_chunks = per_tile // CHUNK
    pad = NW * per_tile - nnz
    rows_p = jnp.pad(rows, (0, pad)).reshape(NW, n_chunks, CHUNK)
    cols_p = jnp.pad(cols, (0, pad)).reshape(NW, n_chunks, CHUNK)
    vals_p = jnp.pad(values, (0, pad)).reshape(NW, n_chunks * CHUNK)

    spmm = functools.partial(
        pl.kernel,
        out_type=jax.ShapeDtypeStruct((NC, cout, b), jnp.float32),
        mesh=plsc.VectorSubcoreMesh(core_axis_name="c", subcore_axis_name="s"),
        scratch_types=[
            pltpu.VMEM((n_chunks, CHUNK), jnp.int32),
            pltpu.VMEM((n_chunks, CHUNK), jnp.int32),
            pltpu.VMEM((n_chunks * CHUNK,), jnp.float32),
            pltpu.VMEM((CHUNK, b), jnp.float32),
            pltpu.VMEM((CHUNK, b), jnp.float32),
            pltpu.VMEM_SHARED((cout, b), jnp.float32),
            pltpu.SemaphoreType.DMA,
            pltpu.SemaphoreType.DMA,
            pltpu.SemaphoreType.DMA,
            pltpu.SemaphoreType.DMA,
        ],
        compiler_params=pltpu.CompilerParams(needs_layout_passes=False,
                                             use_tc_tiling_on_sc=False),
    )(functools.partial(_spmm_sc_kernel, cout, n_chunks))

    partials = spmm(xt, cols_p, rows_p, vals_p)

    n_blk = 8
    merged = pl.pallas_call(
        _merge_body,
        out_shape=jax.ShapeDtypeStruct((cout, b), jnp.float32),
        grid=(n_blk,),
        in_specs=[pl.BlockSpec((NC, cout // n_blk, b), lambda i: (0, i, 0))],
        out_specs=pl.BlockSpec((cout // n_blk, b), lambda i: (i, 0)),
    )(partials)
    return merged.T


# speedup vs baseline: 9.3645x; 1.0206x over previous
import functools

import jax
import jax.numpy as jnp
from jax import lax
from jax.experimental import pallas as pl
from jax.experimental.pallas import tpu as pltpu
from jax.experimental.pallas import tpu_sc as plsc

NC = 2
NS = 16
NL = 16
NW = NC * NS

CHUNK = 128


def _spmm_sc_kernel(cout, n_chunks, xt_hbm, cols_hbm, rows_hbm, vals_hbm,
                    out_hbm, cols_v, rows_v, vals_v, g_a, g_b,
                    accum, sem_ga, sem_gb, sem_sa, sem_sb):
    c = lax.axis_index("c")
    s = lax.axis_index("s")
    wid = c * NS + s
    b = g_a.shape[1]

    zero16 = jnp.zeros((NL,), jnp.float32)

    def _zrow(i, _):
        for q in range(b // NL):
            g_a[i, pl.ds(q * NL, NL)] = zero16
        return 0
    lax.fori_loop(0, CHUNK, _zrow, 0)

    rows_per_tile = cout // NS
    for k in range(rows_per_tile // CHUNK):
        pltpu.sync_copy(g_a, accum.at[pl.ds(s * rows_per_tile + k * CHUNK, CHUNK)])
    plsc.subcore_barrier()

    pltpu.sync_copy(cols_hbm.at[wid], cols_v)
    pltpu.sync_copy(rows_hbm.at[wid], rows_v)
    pltpu.sync_copy(vals_hbm.at[wid], vals_v)

    def _gather(ch, g, sem):
        pltpu.async_copy(xt_hbm.at[cols_v.at[ch]], g, sem)

    def _gather_wait(ch, g, sem):
        pltpu.make_async_copy(xt_hbm.at[cols_v.at[ch]], g, sem).wait()

    def _scatter(ch, g, sem):
        pltpu.async_copy(g, accum.at[rows_v.at[ch]], sem, add=True)

    def _scatter_wait(ch, g, sem):
        pltpu.make_async_copy(g, accum.at[rows_v.at[ch]], sem).wait()

    def _scale(ch, g):
        base16 = jnp.full((NL,), ch * CHUNK, jnp.int32)

        unroll = 8

        def _body(jo, _):
            j0 = jo * unroll
            for u in range(unroll):
                v16 = plsc.load_gather(vals_v, [base16 + (j0 + u)])
                for q in range(b // NL):
                    g[j0 + u, pl.ds(q * NL, NL)] = (
                        g[j0 + u, pl.ds(q * NL, NL)] * v16)
            return 0
        lax.fori_loop(0, CHUNK // unroll, _body, 0)

    _gather(0, g_a, sem_ga)
    _gather(1, g_b, sem_gb)
    last = n_chunks - 1

    def _chunk_body(i, _):
        c0 = 2 * i
        c1 = 2 * i + 1
        _gather_wait(c0, g_a, sem_ga)
        _scale(c0, g_a)
        _scatter(c0, g_a, sem_sa)
        _gather_wait(c1, g_b, sem_gb)
        _scale(c1, g_b)
        _scatter(c1, g_b, sem_sb)
        _scatter_wait(c0, g_a, sem_sa)
        _gather(jnp.minimum(c0 + 2, last), g_a, sem_ga)
        _scatter_wait(c1, g_b, sem_sb)
        _gather(jnp.minimum(c1 + 2, last), g_b, sem_gb)
        return 0

    lax.fori_loop(0, n_chunks // 2, _chunk_body, 0)
    _gather_wait(last, g_a, sem_ga)
    _gather_wait(last, g_b, sem_gb)
    plsc.subcore_barrier()

    pltpu.sync_copy(accum.at[pl.ds(s * rows_per_tile, rows_per_tile)],
                    out_hbm.at[c].at[pl.ds(s * rows_per_tile, rows_per_tile)])


def _merge_body(p_ref, o_ref):
    o_ref[...] = p_ref[0] + p_ref[1]


@jax.jit
def kernel(x, values, indices_float):
    b, cin = x.shape
    nnz = values.shape[0]
    cout = cin

    idx = jnp.round(indices_float).astype(jnp.int32)
    rows, cols = idx[0], idx[1]
    xt = x.T

    per_tile = -(-nnz // NW)
    per_tile = -(-per_tile // CHUNK) * CHUNK
    n_chunks = per_tile // CHUNK
    pad = NW * per_tile - nnz
    rows_p = jnp.pad(rows, (0, pad)).reshape(NW, n_chunks, CHUNK)
    cols_p = jnp.pad(cols, (0, pad)).reshape(NW, n_chunks, CHUNK)
    vals_p = jnp.pad(values, (0, pad)).reshape(NW, n_chunks * CHUNK)

    spmm = functools.partial(
        pl.kernel,
        out_type=jax.ShapeDtypeStruct((NC, cout, b), jnp.float32),
        mesh=plsc.VectorSubcoreMesh(core_axis_name="c", subcore_axis_name="s"),
        scratch_types=[
            pltpu.VMEM((n_chunks, CHUNK), jnp.int32),
            pltpu.VMEM((n_chunks, CHUNK), jnp.int32),
            pltpu.VMEM((n_chunks * CHUNK,), jnp.float32),
            pltpu.VMEM((CHUNK, b), jnp.float32),
            pltpu.VMEM((CHUNK, b), jnp.float32),
            pltpu.VMEM_SHARED((cout, b), jnp.float32),
            pltpu.SemaphoreType.DMA,
            pltpu.SemaphoreType.DMA,
            pltpu.SemaphoreType.DMA,
            pltpu.SemaphoreType.DMA,
        ],
        compiler_params=pltpu.CompilerParams(needs_layout_passes=False,
                                             use_tc_tiling_on_sc=False),
    )(functools.partial(_spmm_sc_kernel, cout, n_chunks))

    partials = spmm(xt, cols_p, rows_p, vals_p)

    n_blk = 8
    merged = pl.pallas_call(
        _merge_body,
        out_shape=jax.ShapeDtypeStruct((cout, b), jnp.float32),
        grid=(n_blk,),
        in_specs=[pl.BlockSpec((NC, cout // n_blk, b), lambda i: (0, i, 0))],
        out_specs=pl.BlockSpec((cout // n_blk, b), lambda i: (i, 0)),
    )(partials)
    return merged.T
